# baseline (speedup 1.0000x reference)
import functools

import jax
import jax.numpy as jnp
from jax import lax
from jax.experimental import pallas as pl
from jax.experimental.pallas import tpu as pltpu
from jax.experimental.pallas import tpu_sc as plsc

NUM_DEGREE = 1000
D_MODEL = 64
N_ROWS, N_COLS = 4096, 200
NC, NS = 2, 16
NW = NC * NS
BW = N_ROWS // NW
V_PAD = 1024
TAB_WORDS = D_MODEL * V_PAD
LANES = 16
N_BG = BW // LANES
NB_OUT = 2


@functools.partial(
    pl.kernel,
    out_type=jax.ShapeDtypeStruct((N_COLS, D_MODEL, N_ROWS), jnp.float32),
    mesh=plsc.VectorSubcoreMesh(core_axis_name="c", subcore_axis_name="s"),
    compiler_params=pltpu.CompilerParams(
        use_tc_tiling_on_sc=True, needs_layout_passes=False),
    scratch_types=[
        pltpu.VMEM((N_COLS, BW), jnp.int32),
        pltpu.VMEM((TAB_WORDS,), jnp.float32),
        pltpu.VMEM((NB_OUT, D_MODEL, BW), jnp.float32),
        pltpu.SemaphoreType.DMA,
        pltpu.SemaphoreType.DMA,
        [pltpu.SemaphoreType.DMA] * NB_OUT,
    ],
)
def _gather_kernel(idx_hbm, tab_hbm, out_hbm, idx_all, table_v, out_v,
                   sem_idx, sem_tab, sem_out):
    wid = lax.axis_index("s") * NC + lax.axis_index("c")
    bw = wid * BW

    cp_idx = pltpu.make_async_copy(
        idx_hbm.at[pl.ds(0, N_COLS), pl.ds(bw, BW)], idx_all, sem_idx)
    cp_tab = pltpu.make_async_copy(tab_hbm, table_v, sem_tab)
    cp_idx.start()
    cp_tab.start()
    cp_idx.wait()
    cp_tab.wait()

    HALF = D_MODEL // 4

    def fire_out_half(b, n, h):
        pltpu.async_copy(
            out_v.at[b].at[pl.ds(h * HALF, HALF), pl.ds(0, BW)],
            out_hbm.at[n].at[pl.ds(h * HALF, HALF), pl.ds(bw, BW)],
            sem_out[b])

    def wait_out(b):
        for h in range(4):
            pltpu.make_async_copy(
                out_v.at[b].at[pl.ds(h * HALF, HALF), pl.ds(0, BW)],
                out_hbm.at[0].at[pl.ds(h * HALF, HALF), pl.ds(bw, BW)],
                sem_out[b]).wait()

    def compute_n(n, b):
        bases = []
        for bg in range(N_BG):
            vi = idx_all[n, pl.ds(bg * LANES, LANES)]
            bases.append(((vi >> 7) << 10) + (vi & 127))
        fire_at = HALF * N_BG - 1
        pend, stores = [], 0
        for c in range(D_MODEL):
            for bg in range(N_BG):
                kc = (c // 8) * 8192 + (c % 8) * 128
                g = plsc.load_gather(
                    table_v.at[pl.ds(kc, TAB_WORDS - kc)], [bases[bg]])
                pend.append((bg, c, g))
                if len(pend) > LANES:
                    pbg, pc, pg = pend.pop(0)
                    out_v[b, pc, pl.ds(pbg * LANES, LANES)] = pg
                    stores += 1
                    if stores % (HALF * N_BG) == 0 and stores // (HALF * N_BG) <= 3:
                        fire_out_half(b, n, stores // (HALF * N_BG) - 1)
        for pbg, pc, pg in pend:
            out_v[b, pc, pl.ds(pbg * LANES, LANES)] = pg

    def pair_body(p, carry):
        for r in range(NB_OUT):
            n = p * NB_OUT + r
            pl.when(p > 0)(lambda r=r: wait_out(r))
            compute_n(n, r)
            fire_out_half(r, n, 3)
        return carry

    lax.fori_loop(0, N_COLS // NB_OUT, pair_body, 0)
    for b in range(NB_OUT):
        wait_out(b)


def kernel(degree, degree_encoder_weight):
    idx_t = degree.T
    tab_t = degree_encoder_weight.T
    tab_pad = jnp.pad(tab_t, ((0, 0), (0, V_PAD - NUM_DEGREE)))
    tab_flat = (tab_pad.reshape(8, 8, 8, 128)
                .transpose(0, 2, 1, 3).reshape(TAB_WORDS))
    res = _gather_kernel(idx_t, tab_flat)
    return res.transpose(2, 0, 1)


# speedup vs baseline: 1.0018x; 1.0018x over previous
import functools

import jax
import jax.numpy as jnp
from jax import lax
from jax.experimental import pallas as pl
from jax.experimental.pallas import tpu as pltpu
from jax.experimental.pallas import tpu_sc as plsc

NUM_DEGREE = 1000
D_MODEL = 64
N_ROWS, N_COLS = 4096, 200
NC, NS = 2, 16
NW = NC * NS
BW = N_ROWS // NW
V_PAD = 1024
TAB_WORDS = D_MODEL * V_PAD
LANES = 16
N_BG = BW // LANES
NB_OUT = 2


@functools.partial(
    pl.kernel,
    out_type=jax.ShapeDtypeStruct((N_COLS, D_MODEL, N_ROWS), jnp.float32),
    mesh=plsc.VectorSubcoreMesh(core_axis_name="c", subcore_axis_name="s"),
    compiler_params=pltpu.CompilerParams(
        use_tc_tiling_on_sc=True, needs_layout_passes=False),
    scratch_types=[
        pltpu.VMEM((N_COLS, BW), jnp.int32),
        pltpu.VMEM((TAB_WORDS,), jnp.float32),
        pltpu.VMEM((NB_OUT, D_MODEL, BW), jnp.float32),
        pltpu.SemaphoreType.DMA,
        pltpu.SemaphoreType.DMA,
        [pltpu.SemaphoreType.DMA] * NB_OUT,
    ],
)
def _gather_kernel(idx_hbm, tab_hbm, out_hbm, idx_all, table_v, out_v,
                   sem_idx, sem_tab, sem_out):
    wid = lax.axis_index("s") * NC + lax.axis_index("c")
    bw = wid * BW

    cp_idx = pltpu.make_async_copy(
        idx_hbm.at[pl.ds(0, N_COLS), pl.ds(bw, BW)], idx_all, sem_idx)
    cp_tab = pltpu.make_async_copy(tab_hbm, table_v, sem_tab)
    cp_idx.start()
    cp_tab.start()
    cp_idx.wait()
    cp_tab.wait()

    HALF = D_MODEL // 2

    def fire_out_half(b, n, h):
        pltpu.async_copy(
            out_v.at[b].at[pl.ds(h * HALF, HALF), pl.ds(0, BW)],
            out_hbm.at[n].at[pl.ds(h * HALF, HALF), pl.ds(bw, BW)],
            sem_out[b])

    def wait_out(b):
        for h in range(2):
            pltpu.make_async_copy(
                out_v.at[b].at[pl.ds(h * HALF, HALF), pl.ds(0, BW)],
                out_hbm.at[0].at[pl.ds(h * HALF, HALF), pl.ds(bw, BW)],
                sem_out[b]).wait()

    def compute_n(n, b):
        bases = []
        for bg in range(N_BG):
            vi = idx_all[n, pl.ds(bg * LANES, LANES)]
            bases.append(((vi >> 7) << 10) + (vi & 127))
        fire_at = HALF * N_BG - 1
        pend, stores = [], 0
        for c in range(D_MODEL):
            for bg in range(N_BG):
                kc = (c // 8) * 8192 + (c % 8) * 128
                g = plsc.load_gather(
                    table_v.at[pl.ds(kc, TAB_WORDS - kc)], [bases[bg]])
                pend.append((bg, c, g))
                if len(pend) > LANES:
                    pbg, pc, pg = pend.pop(0)
                    out_v[b, pc, pl.ds(pbg * LANES, LANES)] = pg
                    stores += 1
                    if stores == fire_at + 1:
                        fire_out_half(b, n, 0)
        for pbg, pc, pg in pend:
            out_v[b, pc, pl.ds(pbg * LANES, LANES)] = pg

    def pair_body(p, carry):
        for r in range(NB_OUT):
            n = p * NB_OUT + r
            pl.when(p > 0)(lambda r=r: wait_out(r))
            compute_n(n, r)
            fire_out_half(r, n, 1)
        return carry

    lax.fori_loop(0, N_COLS // NB_OUT, pair_body, 0)
    for b in range(NB_OUT):
        wait_out(b)


def kernel(degree, degree_encoder_weight):
    idx_t = degree.T
    tab_t = degree_encoder_weight.T
    tab_pad = jnp.pad(tab_t, ((0, 0), (0, V_PAD - NUM_DEGREE)))
    tab_flat = (tab_pad.reshape(8, 8, 8, 128)
                .transpose(0, 2, 1, 3).reshape(TAB_WORDS))
    res = _gather_kernel(idx_t, tab_flat)
    return res.transpose(2, 0, 1)
